# baseline (speedup 1.0000x reference)
import jax
import jax.numpy as jnp
from jax.experimental import pallas as pl
from jax.experimental.pallas import tpu as pltpu

RANK = 4
ALPHA = 1.0
NSUB = 16
DIN = 1024
DFF = 4096
TS = 512
CHUNK = 256


def _fold(sid_ref, W1_ref, A1_ref, B1_ref, W2_ref, A2_ref, B2_ref,
          W1e_ref, W2e_ref):
    scale = ALPHA / RANK
    a1 = (A1_ref[0].astype(jnp.float32) * scale).astype(jnp.bfloat16)
    d1 = jnp.dot(a1, B1_ref[0], preferred_element_type=jnp.float32)
    W1e_ref[0] = W1_ref[...] + d1.astype(jnp.bfloat16)
    a2 = (A2_ref[0].astype(jnp.float32) * (0.5 * scale)).astype(jnp.bfloat16)
    d2 = jnp.dot(a2, B2_ref[0], preferred_element_type=jnp.float32)
    W2e_ref[0] = jnp.float32(0.5).astype(jnp.bfloat16) * W2_ref[...] \
        + d2.astype(jnp.bfloat16)


def _mlp(x_ref, W1e_ref, b1_ref, W2e_ref, b2_ref, out_ref):
    for c in range(TS // CHUNK):
        sl = pl.ds(c * CHUNK, CHUNK)
        x = x_ref[0, sl, :]
        h = jnp.dot(x, W1e_ref[0], preferred_element_type=jnp.float32)
        h = h + b1_ref[...]
        h = h + h * jax.lax.erf(h * 0.7071067811865476)
        out = jnp.dot(h.astype(jnp.bfloat16), W2e_ref[0],
                      preferred_element_type=jnp.float32)
        out_ref[0, sl, :] = out + b2_ref[...]


def kernel(x, subject_id, W1, b1, A1, B1, W2, b2, A2, B2):
    B, S, _ = x.shape
    bf = jnp.bfloat16
    xb = x.astype(bf)
    A1b, B1b = A1.astype(bf), B1.astype(bf)
    A2b, B2b = A2.astype(bf), B2.astype(bf)
    b1r = b1.reshape(1, DFF)
    b2r = b2.reshape(1, DIN)
    sid = subject_id.astype(jnp.int32)

    fold_spec = pltpu.PrefetchScalarGridSpec(
        num_scalar_prefetch=1,
        grid=(B,),
        in_specs=[
            pl.BlockSpec((DIN, DFF), lambda b, sid: (0, 0)),
            pl.BlockSpec((1, DIN, RANK), lambda b, sid: (sid[b], 0, 0)),
            pl.BlockSpec((1, RANK, DFF), lambda b, sid: (sid[b], 0, 0)),
            pl.BlockSpec((DFF, DIN), lambda b, sid: (0, 0)),
            pl.BlockSpec((1, DFF, RANK), lambda b, sid: (sid[b], 0, 0)),
            pl.BlockSpec((1, RANK, DIN), lambda b, sid: (sid[b], 0, 0)),
        ],
        out_specs=[
            pl.BlockSpec((1, DIN, DFF), lambda b, sid: (b, 0, 0)),
            pl.BlockSpec((1, DFF, DIN), lambda b, sid: (b, 0, 0)),
        ],
    )
    W1e, W2e = pl.pallas_call(
        _fold,
        grid_spec=fold_spec,
        out_shape=[
            jax.ShapeDtypeStruct((B, DIN, DFF), bf),
            jax.ShapeDtypeStruct((B, DFF, DIN), bf),
        ],
        compiler_params=pltpu.CompilerParams(
            dimension_semantics=("arbitrary",),
        ),
    )(sid, W1.astype(bf), A1b, B1b, W2.astype(bf), A2b, B2b)

    out = pl.pallas_call(
        _mlp,
        grid=(B, S // TS),
        in_specs=[
            pl.BlockSpec((1, TS, DIN), lambda b, t: (b, t, 0)),
            pl.BlockSpec((1, DIN, DFF), lambda b, t: (b, 0, 0)),
            pl.BlockSpec((1, DFF), lambda b, t: (0, 0)),
            pl.BlockSpec((1, DFF, DIN), lambda b, t: (b, 0, 0)),
            pl.BlockSpec((1, DIN), lambda b, t: (0, 0)),
        ],
        out_specs=pl.BlockSpec((1, TS, DIN), lambda b, t: (b, t, 0)),
        out_shape=jax.ShapeDtypeStruct((B, S, DIN), jnp.float32),
        compiler_params=pltpu.CompilerParams(
            dimension_semantics=("arbitrary", "arbitrary"),
        ),
    )(xb, W1e, b1r, W2e, b2r)
    return out


# speedup vs baseline: 1.0144x; 1.0144x over previous
import jax
import jax.numpy as jnp
from jax.experimental import pallas as pl
from jax.experimental.pallas import tpu as pltpu

RANK = 4
ALPHA = 1.0
NSUB = 16
DIN = 1024
DFF = 4096
TS = 512
CHUNK = 512


def _fold(sid_ref, W1_ref, A1_ref, B1_ref, W2_ref, A2_ref, B2_ref,
          W1e_ref, W2e_ref):
    scale = ALPHA / RANK
    a1 = (A1_ref[0].astype(jnp.float32) * scale).astype(jnp.bfloat16)
    d1 = jnp.dot(a1, B1_ref[0], preferred_element_type=jnp.float32)
    W1e_ref[0] = W1_ref[...] + d1.astype(jnp.bfloat16)
    a2 = (A2_ref[0].astype(jnp.float32) * (0.5 * scale)).astype(jnp.bfloat16)
    d2 = jnp.dot(a2, B2_ref[0], preferred_element_type=jnp.float32)
    W2e_ref[0] = jnp.float32(0.5).astype(jnp.bfloat16) * W2_ref[...] \
        + d2.astype(jnp.bfloat16)


def _mlp(x_ref, W1e_ref, b1_ref, W2e_ref, b2_ref, out_ref):
    for c in range(TS // CHUNK):
        sl = pl.ds(c * CHUNK, CHUNK)
        x = x_ref[0, sl, :]
        h = jnp.dot(x, W1e_ref[0], preferred_element_type=jnp.float32)
        h = h + b1_ref[...]
        h = h + h * jax.lax.erf(h * 0.7071067811865476)
        out = jnp.dot(h.astype(jnp.bfloat16), W2e_ref[0],
                      preferred_element_type=jnp.float32)
        out_ref[0, sl, :] = out + b2_ref[...]


def kernel(x, subject_id, W1, b1, A1, B1, W2, b2, A2, B2):
    B, S, _ = x.shape
    bf = jnp.bfloat16
    xb = x.astype(bf)
    A1b, B1b = A1.astype(bf), B1.astype(bf)
    A2b, B2b = A2.astype(bf), B2.astype(bf)
    b1r = b1.reshape(1, DFF)
    b2r = b2.reshape(1, DIN)
    sid = subject_id.astype(jnp.int32)

    fold_spec = pltpu.PrefetchScalarGridSpec(
        num_scalar_prefetch=1,
        grid=(B,),
        in_specs=[
            pl.BlockSpec((DIN, DFF), lambda b, sid: (0, 0)),
            pl.BlockSpec((1, DIN, RANK), lambda b, sid: (sid[b], 0, 0)),
            pl.BlockSpec((1, RANK, DFF), lambda b, sid: (sid[b], 0, 0)),
            pl.BlockSpec((DFF, DIN), lambda b, sid: (0, 0)),
            pl.BlockSpec((1, DFF, RANK), lambda b, sid: (sid[b], 0, 0)),
            pl.BlockSpec((1, RANK, DIN), lambda b, sid: (sid[b], 0, 0)),
        ],
        out_specs=[
            pl.BlockSpec((1, DIN, DFF), lambda b, sid: (b, 0, 0)),
            pl.BlockSpec((1, DFF, DIN), lambda b, sid: (b, 0, 0)),
        ],
    )
    W1e, W2e = pl.pallas_call(
        _fold,
        grid_spec=fold_spec,
        out_shape=[
            jax.ShapeDtypeStruct((B, DIN, DFF), bf),
            jax.ShapeDtypeStruct((B, DFF, DIN), bf),
        ],
        compiler_params=pltpu.CompilerParams(
            dimension_semantics=("arbitrary",),
        ),
    )(sid, W1.astype(bf), A1b, B1b, W2.astype(bf), A2b, B2b)

    out = pl.pallas_call(
        _mlp,
        grid=(B, S // TS),
        in_specs=[
            pl.BlockSpec((1, TS, DIN), lambda b, t: (b, t, 0)),
            pl.BlockSpec((1, DIN, DFF), lambda b, t: (b, 0, 0)),
            pl.BlockSpec((1, DFF), lambda b, t: (0, 0)),
            pl.BlockSpec((1, DFF, DIN), lambda b, t: (b, 0, 0)),
            pl.BlockSpec((1, DIN), lambda b, t: (0, 0)),
        ],
        out_specs=pl.BlockSpec((1, TS, DIN), lambda b, t: (b, t, 0)),
        out_shape=jax.ShapeDtypeStruct((B, S, DIN), jnp.float32),
        compiler_params=pltpu.CompilerParams(
            dimension_semantics=("arbitrary", "arbitrary"),
        ),
    )(xb, W1e, b1r, W2e, b2r)
    return out


# speedup vs baseline: 1.1133x; 1.0975x over previous
import jax
import jax.numpy as jnp
from jax.experimental import pallas as pl
from jax.experimental.pallas import tpu as pltpu

RANK = 4
ALPHA = 1.0
NSUB = 16
DIN = 1024
DFF = 4096
TS = 512


def _fold1(sid_ref, W_ref, A_ref, B_ref, We_ref):
    scale = ALPHA / RANK
    a = (A_ref[0] * scale).astype(jnp.bfloat16)
    d = jnp.dot(a, B_ref[0].astype(jnp.bfloat16),
                preferred_element_type=jnp.float32)
    We_ref[0] = (W_ref[...] + d).astype(jnp.bfloat16)


def _fold2(sid_ref, W_ref, A_ref, B_ref, We_ref):
    scale = ALPHA / RANK
    a = (A_ref[0] * (0.5 * scale)).astype(jnp.bfloat16)
    d = jnp.dot(a, B_ref[0].astype(jnp.bfloat16),
                preferred_element_type=jnp.float32)
    We_ref[0] = (0.5 * W_ref[...] + d).astype(jnp.bfloat16)


def _mlp(x_ref, W1e_ref, b1_ref, W2e_ref, b2_ref, out_ref):
    x = x_ref[0].astype(jnp.bfloat16)
    h = jnp.dot(x, W1e_ref[0], preferred_element_type=jnp.float32)
    h = h + b1_ref[...]
    h = h + h * jax.lax.erf(h * 0.7071067811865476)
    out = jnp.dot(h.astype(jnp.bfloat16), W2e_ref[0],
                  preferred_element_type=jnp.float32)
    out_ref[0] = out + b2_ref[...]


def _fold_call(body, W, A, Bk, sid, din, dff):
    B = sid.shape[0]
    spec = pltpu.PrefetchScalarGridSpec(
        num_scalar_prefetch=1,
        grid=(B,),
        in_specs=[
            pl.BlockSpec((din, dff), lambda b, sid: (0, 0)),
            pl.BlockSpec((1, din, RANK), lambda b, sid: (sid[b], 0, 0)),
            pl.BlockSpec((1, RANK, dff), lambda b, sid: (sid[b], 0, 0)),
        ],
        out_specs=pl.BlockSpec((1, din, dff), lambda b, sid: (b, 0, 0)),
    )
    return pl.pallas_call(
        body,
        grid_spec=spec,
        out_shape=jax.ShapeDtypeStruct((B, din, dff), jnp.bfloat16),
        compiler_params=pltpu.CompilerParams(
            dimension_semantics=("arbitrary",),
        ),
    )(sid, W, A, Bk)


def kernel(x, subject_id, W1, b1, A1, B1, W2, b2, A2, B2):
    B, S, _ = x.shape
    b1r = b1.reshape(1, DFF)
    b2r = b2.reshape(1, DIN)
    sid = subject_id.astype(jnp.int32)

    W1e = _fold_call(_fold1, W1, A1, B1, sid, DIN, DFF)
    W2e = _fold_call(_fold2, W2, A2, B2, sid, DFF, DIN)

    out = pl.pallas_call(
        _mlp,
        grid=(B, S // TS),
        in_specs=[
            pl.BlockSpec((1, TS, DIN), lambda b, t: (b, t, 0)),
            pl.BlockSpec((1, DIN, DFF), lambda b, t: (b, 0, 0)),
            pl.BlockSpec((1, DFF), lambda b, t: (0, 0)),
            pl.BlockSpec((1, DFF, DIN), lambda b, t: (b, 0, 0)),
            pl.BlockSpec((1, DIN), lambda b, t: (0, 0)),
        ],
        out_specs=pl.BlockSpec((1, TS, DIN), lambda b, t: (b, t, 0)),
        out_shape=jax.ShapeDtypeStruct((B, S, DIN), jnp.float32),
        compiler_params=pltpu.CompilerParams(
            dimension_semantics=("arbitrary", "arbitrary"),
        ),
    )(x, W1e, b1r, W2e, b2r)
    return out
